# baseline (speedup 1.0000x reference)
import functools

import jax
import jax.numpy as jnp
from jax import lax
from jax.experimental import pallas as pl
from jax.experimental.pallas import tpu as pltpu
from jax.experimental.pallas import tpu_sc as plsc

_F32 = jnp.float32
_BF16 = jnp.bfloat16
_I32 = jnp.int32

_NC = 2
_NSUB = 16
_NW = _NC * _NSUB


def _gelu_exact(x):
    return 0.5 * x * (1.0 + jax.lax.erf(x * 0.7071067811865476))


def _sc_gather(hl2d, flat_idx):
    R = flat_idx.shape[0]
    D = hl2d.shape[1]
    rows_per = R // _NW
    mesh = plsc.VectorSubcoreMesh(core_axis_name="c", subcore_axis_name="s")

    @functools.partial(
        pl.kernel,
        out_type=jax.ShapeDtypeStruct((R, D), _F32),
        mesh=mesh,
        scratch_types=[
            pltpu.VMEM((rows_per,), _I32),
            pltpu.VMEM((rows_per, D), _F32),
            pltpu.SemaphoreType.DMA,
        ],
    )
    def k(hl_hbm, idx_hbm, out_hbm, idx_v, rows_v, sem):
        wid = lax.axis_index("s") * _NC + lax.axis_index("c")
        base = wid * rows_per
        pltpu.sync_copy(idx_hbm.at[pl.ds(base, rows_per)], idx_v)
        pltpu.async_copy(hl_hbm.at[idx_v], rows_v, sem).wait()
        pltpu.sync_copy(rows_v, out_hbm.at[pl.ds(base, rows_per)])

    return k(hl2d, flat_idx)


def _sc_scatter(ln2d, sidx, zrows, NR):
    R, D = ln2d.shape
    ZR = zrows.shape[0]
    sp = NR // _NW
    rp = R // _NW
    mesh = plsc.VectorSubcoreMesh(core_axis_name="c", subcore_axis_name="s")

    @functools.partial(
        pl.kernel,
        out_type=jax.ShapeDtypeStruct((NR, D), _F32),
        mesh=mesh,
        scratch_types=[
            pltpu.VMEM((ZR, D), _F32),
            pltpu.VMEM((rp,), _I32),
            pltpu.VMEM((rp, D), _F32),
            pltpu.SemaphoreType.DMA,
        ],
    )
    def k(ln_hbm, sidx_hbm, z_hbm, out_hbm, zbuf, idx_v, rows_v, sem):
        c = lax.axis_index("c")
        s = lax.axis_index("s")
        base = (c * _NSUB + s) * sp
        pltpu.sync_copy(z_hbm, zbuf)
        for j in range(sp // ZR):
            pltpu.sync_copy(zbuf, out_hbm.at[pl.ds(base + j * ZR, ZR)])
        plsc.subcore_barrier()
        r0 = (c * _NSUB + s) * rp
        pltpu.sync_copy(sidx_hbm.at[pl.ds(r0, rp)], idx_v)
        pltpu.sync_copy(ln_hbm.at[pl.ds(r0, rp)], rows_v)
        pltpu.async_copy(rows_v, out_hbm.at[idx_v], sem).wait()

    return k(ln2d, sidx, zrows)


def _moe_body(hu_ref, hm_ref, mc_ref, ur_ref, r_ref, wr_ref, br_ref,
              w1_ref, b1_ref, w2_ref, b2_ref, ln_ref,
              w_s, adjn_s, cpos_s, mix_s, *, B, U, M, K):
    k = pl.program_id(0)
    BM = B * M
    BU = B * U

    @pl.when(k == 0)
    def _init():
        logits = jnp.dot(hm_ref[...], wr_ref[...], preferred_element_type=_F32)
        logits = logits + br_ref[...]
        mx = jnp.max(logits, axis=-1, keepdims=True)
        e = jnp.exp(logits - mx)
        w_s[...] = e / jnp.sum(e, axis=-1, keepdims=True)
        diff = jnp.abs(ur_ref[...] - mc_ref[...])
        same_b = (jax.lax.broadcasted_iota(_I32, (BM, BU), 0) // M
                  == jax.lax.broadcasted_iota(_I32, (BM, BU), 1) // U)
        adj = ((diff > 0) & (diff <= r_ref[0]) & same_b).astype(_F32)
        cnt = jnp.sum(adj, axis=-1, keepdims=True)
        adjn_s[...] = adj / jnp.maximum(cnt, 1.0)
        cpos_s[...] = (cnt > 0.0).astype(_F32)
        mix_s[...] = jnp.dot(w_s[...], b2_ref[...], preferred_element_type=_F32)

    hid = jnp.dot(hu_ref[...], w1_ref[0].astype(_BF16),
                  preferred_element_type=_F32)
    hid = _gelu_exact(hid + b1_ref[0]).astype(_BF16)
    sel = (jax.lax.broadcasted_iota(_I32, (1, K), 1) == k).astype(_F32)
    w_col = jnp.sum(w_s[...] * sel, axis=-1, keepdims=True)
    aw = (adjn_s[...] * w_col).astype(_BF16)
    t = jnp.dot(aw, hid, preferred_element_type=_F32).astype(_BF16)
    mix_s[...] += jnp.dot(t, w2_ref[0].astype(_BF16),
                          preferred_element_type=_F32)

    @pl.when(k == K - 1)
    def _fin():
        mix = mix_s[...]
        mu = jnp.mean(mix, axis=-1, keepdims=True)
        var = jnp.mean((mix - mu) ** 2, axis=-1, keepdims=True)
        ln_ref[...] = (mix - mu) * jax.lax.rsqrt(var + 1e-5) * cpos_s[...]


def kernel(h_L, mask_indices, unmasked_indices, range_r, W_r, b_r,
           W1, b1, W2, b2):
    B, S, D = h_L.shape
    M = mask_indices.shape[1]
    U = unmasked_indices.shape[1]
    K = W_r.shape[1]
    F = W1.shape[2]
    mi = mask_indices.astype(_I32)
    ui = unmasked_indices.astype(_I32)
    r_arr = jnp.asarray(range_r, _I32).reshape(1)

    offs = (jnp.arange(B, dtype=_I32) * S)[:, None]
    flat_idx = jnp.concatenate(
        [(ui + offs).reshape(-1), (mi + offs).reshape(-1)])
    g = _sc_gather(h_L.reshape(B * S, D), flat_idx)
    hu = g[:B * U].astype(_BF16)
    hm = g[B * U:]

    ln = pl.pallas_call(
        functools.partial(_moe_body, B=B, U=U, M=M, K=K),
        grid=(K,),
        in_specs=[
            pl.BlockSpec((B * U, D), lambda k: (0, 0)),
            pl.BlockSpec((B * M, D), lambda k: (0, 0)),
            pl.BlockSpec((B * M, 1), lambda k: (0, 0)),
            pl.BlockSpec((1, B * U), lambda k: (0, 0)),
            pl.BlockSpec(memory_space=pltpu.SMEM),
            pl.BlockSpec((D, K), lambda k: (0, 0)),
            pl.BlockSpec((1, K), lambda k: (0, 0)),
            pl.BlockSpec((1, D, F), lambda k: (k, 0, 0)),
            pl.BlockSpec((1, 1, F), lambda k: (k, 0, 0)),
            pl.BlockSpec((1, F, D), lambda k: (k, 0, 0)),
            pl.BlockSpec((K, D), lambda k: (0, 0)),
        ],
        out_specs=pl.BlockSpec((B * M, D), lambda k: (0, 0)),
        out_shape=jax.ShapeDtypeStruct((B * M, D), _F32),
        scratch_shapes=[
            pltpu.VMEM((B * M, K), _F32),
            pltpu.VMEM((B * M, B * U), _F32),
            pltpu.VMEM((B * M, 1), _F32),
            pltpu.VMEM((B * M, D), _F32),
        ],
    )(hu, hm, mi.reshape(B * M, 1), ui.reshape(1, B * U), r_arr,
      W_r, b_r.reshape(1, K), W1, b1.reshape(K, 1, F), W2, b2)

    sidx = (mi + offs).reshape(-1)
    zrows = jnp.zeros((32, D), _F32)
    delta = _sc_scatter(ln, sidx, zrows, B * S)
    return delta.reshape(B, S, D)


# speedup vs baseline: 1.1533x; 1.1533x over previous
import functools

import jax
import jax.numpy as jnp
from jax import lax
from jax.experimental import pallas as pl
from jax.experimental.pallas import tpu as pltpu
from jax.experimental.pallas import tpu_sc as plsc

_F32 = jnp.float32
_BF16 = jnp.bfloat16
_I32 = jnp.int32

_NC = 2
_NSUB = 16
_NW = _NC * _NSUB


def _gelu_exact(x):
    return 0.5 * x * (1.0 + jax.lax.erf(x * 0.7071067811865476))


def _sc_gather(hl2d, flat_idx):
    R = flat_idx.shape[0]
    D = hl2d.shape[1]
    rows_per = R // _NW
    mesh = plsc.VectorSubcoreMesh(core_axis_name="c", subcore_axis_name="s")

    @functools.partial(
        pl.kernel,
        out_type=jax.ShapeDtypeStruct((R, D), _F32),
        mesh=mesh,
        scratch_types=[
            pltpu.VMEM((rows_per,), _I32),
            pltpu.VMEM((rows_per, D), _F32),
            pltpu.SemaphoreType.DMA,
        ],
    )
    def k(hl_hbm, idx_hbm, out_hbm, idx_v, rows_v, sem):
        wid = lax.axis_index("s") * _NC + lax.axis_index("c")
        base = wid * rows_per
        pltpu.sync_copy(idx_hbm.at[pl.ds(base, rows_per)], idx_v)
        pltpu.async_copy(hl_hbm.at[idx_v], rows_v, sem).wait()
        pltpu.sync_copy(rows_v, out_hbm.at[pl.ds(base, rows_per)])

    return k(hl2d, flat_idx)


def _tc_body(hu_ref, hm_ref, mc_ref, mr_ref, ur_ref, r_ref, wr_ref, br_ref,
             w1_ref, b1_ref, w2_ref, b2_ref, out_ref,
             w_s, adjn_s, cpos_s, mix_s, *, B, NS, ST, U, M, K):
    p = pl.program_id(0)
    BM = B * M
    BU = B * U

    @pl.when(p == 0)
    def _init():
        logits = jnp.dot(hm_ref[...], wr_ref[...], preferred_element_type=_F32)
        logits = logits + br_ref[...]
        mx = jnp.max(logits, axis=-1, keepdims=True)
        e = jnp.exp(logits - mx)
        w_s[...] = e / jnp.sum(e, axis=-1, keepdims=True)
        diff = jnp.abs(ur_ref[...] - mc_ref[...])
        same_b = (jax.lax.broadcasted_iota(_I32, (BM, BU), 0) // M
                  == jax.lax.broadcasted_iota(_I32, (BM, BU), 1) // U)
        adj = ((diff > 0) & (diff <= r_ref[0]) & same_b).astype(_F32)
        cnt = jnp.sum(adj, axis=-1, keepdims=True)
        adjn_s[...] = adj / jnp.maximum(cnt, 1.0)
        cpos_s[...] = (cnt > 0.0).astype(_F32)
        mix_s[...] = jnp.dot(w_s[...], b2_ref[...], preferred_element_type=_F32)

    @pl.when(p < K)
    def _expert():
        hid = jnp.dot(hu_ref[...], w1_ref[0].astype(_BF16),
                      preferred_element_type=_F32)
        hid = _gelu_exact(hid + b1_ref[0]).astype(_BF16)
        sel = (jax.lax.broadcasted_iota(_I32, (1, K), 1) == p).astype(_F32)
        w_col = jnp.sum(w_s[...] * sel, axis=-1, keepdims=True)
        aw = (adjn_s[...] * w_col).astype(_BF16)
        t = jnp.dot(aw, hid, preferred_element_type=_F32).astype(_BF16)
        mix_s[...] += jnp.dot(t, w2_ref[0].astype(_BF16),
                              preferred_element_type=_F32)

    @pl.when(p == K - 1)
    def _fin():
        mix = mix_s[...]
        mu = jnp.mean(mix, axis=-1, keepdims=True)
        var = jnp.mean((mix - mu) ** 2, axis=-1, keepdims=True)
        mix_s[...] = (mix - mu) * jax.lax.rsqrt(var + 1e-5) * cpos_s[...]

    @pl.when(p >= K)
    def _scatter():
        q = p - K
        b = q // NS
        s = q % NS
        base = s * ST
        mr = mr_ref[0]
        nxt = jnp.concatenate([mr[:, 1:], jnp.full((1, 1), -1, _I32)], axis=1)
        last = mr != nxt
        col = jax.lax.broadcasted_iota(_I32, (ST, M), 0) + base
        pm = ((col == mr) & last).astype(_F32)
        mb = pl.multiple_of(b * M, M)
        out_ref[0] = jnp.dot(pm, mix_s[pl.ds(mb, M)],
                             preferred_element_type=_F32)


def kernel(h_L, mask_indices, unmasked_indices, range_r, W_r, b_r,
           W1, b1, W2, b2):
    B, S, D = h_L.shape
    M = mask_indices.shape[1]
    U = unmasked_indices.shape[1]
    K = W_r.shape[1]
    F = W1.shape[2]
    ST = 512
    NS = S // ST
    NSB = B * NS
    mi = mask_indices.astype(_I32)
    ui = unmasked_indices.astype(_I32)
    r_arr = jnp.asarray(range_r, _I32).reshape(1)

    offs = (jnp.arange(B, dtype=_I32) * S)[:, None]
    flat_idx = jnp.concatenate(
        [(ui + offs).reshape(-1), (mi + offs).reshape(-1)])
    g = _sc_gather(h_L.reshape(B * S, D), flat_idx)
    hu = g[:B * U].astype(_BF16)
    hm = g[B * U:]

    def w_idx(p):
        return (jnp.clip(p, 0, K - 1), 0, 0)

    def mr_idx(p):
        return (jnp.clip((p - K) // NS, 0, B - 1), 0, 0)

    def out_idx(p):
        q = jnp.clip(p - K, 0, NSB - 1)
        return (q // NS, q % NS, 0)

    out = pl.pallas_call(
        functools.partial(_tc_body, B=B, NS=NS, ST=ST, U=U, M=M, K=K),
        grid=(K + NSB,),
        in_specs=[
            pl.BlockSpec((B * U, D), lambda p: (0, 0)),
            pl.BlockSpec((B * M, D), lambda p: (0, 0)),
            pl.BlockSpec((B * M, 1), lambda p: (0, 0)),
            pl.BlockSpec((1, 1, M), mr_idx),
            pl.BlockSpec((1, B * U), lambda p: (0, 0)),
            pl.BlockSpec(memory_space=pltpu.SMEM),
            pl.BlockSpec((D, K), lambda p: (0, 0)),
            pl.BlockSpec((1, K), lambda p: (0, 0)),
            pl.BlockSpec((1, D, F), w_idx),
            pl.BlockSpec((1, 1, F), w_idx),
            pl.BlockSpec((1, F, D), w_idx),
            pl.BlockSpec((K, D), lambda p: (0, 0)),
        ],
        out_specs=pl.BlockSpec((1, ST, D), out_idx),
        out_shape=jax.ShapeDtypeStruct((B, S, D), _F32),
        scratch_shapes=[
            pltpu.VMEM((B * M, K), _F32),
            pltpu.VMEM((B * M, B * U), _F32),
            pltpu.VMEM((B * M, 1), _F32),
            pltpu.VMEM((B * M, D), _F32),
        ],
    )(hu, hm, mi.reshape(B * M, 1), mi.reshape(B, 1, M),
      ui.reshape(1, B * U), r_arr,
      W_r, b_r.reshape(1, K), W1, b1.reshape(K, 1, F), W2, b2)
    return out


# speedup vs baseline: 1.2447x; 1.0792x over previous
import functools

import jax
import jax.numpy as jnp
from jax import lax
from jax.experimental import pallas as pl
from jax.experimental.pallas import tpu as pltpu
from jax.experimental.pallas import tpu_sc as plsc

_F32 = jnp.float32
_BF16 = jnp.bfloat16
_I32 = jnp.int32

_NC = 2
_NSUB = 16
_NW = _NC * _NSUB


def _gelu_exact(x):
    return 0.5 * x * (1.0 + jax.lax.erf(x * 0.7071067811865476))


def _sc_gather(hl2d, uidx, midx):
    RU = uidx.shape[0]
    RM = midx.shape[0]
    D = hl2d.shape[1]
    upw = RU // _NW
    mpw = RM // _NW
    mesh = plsc.VectorSubcoreMesh(core_axis_name="c", subcore_axis_name="s")

    @functools.partial(
        pl.kernel,
        out_type=[
            jax.ShapeDtypeStruct((RU, D), _F32),
            jax.ShapeDtypeStruct((RM, D), _F32),
        ],
        mesh=mesh,
        scratch_types=[
            pltpu.VMEM((upw,), _I32),
            pltpu.VMEM((upw, D), _F32),
            pltpu.VMEM((mpw,), _I32),
            pltpu.VMEM((mpw, D), _F32),
            pltpu.SemaphoreType.DMA,
        ],
    )
    def k(hl_hbm, uidx_hbm, midx_hbm, hu_hbm, hm_hbm,
          uidx_v, urows_v, midx_v, mrows_v, sem):
        wid = lax.axis_index("s") * _NC + lax.axis_index("c")
        ub = wid * upw
        mb = wid * mpw
        pltpu.sync_copy(uidx_hbm.at[pl.ds(ub, upw)], uidx_v)
        pltpu.sync_copy(midx_hbm.at[pl.ds(mb, mpw)], midx_v)
        cp_u = pltpu.async_copy(hl_hbm.at[uidx_v], urows_v, sem)
        cp_m = pltpu.async_copy(hl_hbm.at[midx_v], mrows_v, sem)
        cp_u.wait()
        cp_m.wait()
        pltpu.sync_copy(urows_v, hu_hbm.at[pl.ds(ub, upw)])
        pltpu.sync_copy(mrows_v, hm_hbm.at[pl.ds(mb, mpw)])

    return k(hl2d, uidx, midx)


def _tc_body(hu_ref, hm_ref, mc_ref, mr_ref, ur_ref, r_ref, wr_ref, br_ref,
             w1_ref, b1_ref, w2_ref, b2_ref, out_ref,
             hub_s, w_s, adjn_s, cpos_s, mix_s, *, B, NS, ST, U, M, K):
    p = pl.program_id(0)
    BM = B * M
    BU = B * U

    @pl.when(p == 0)
    def _init():
        hub_s[...] = hu_ref[...].astype(_BF16)
        logits = jnp.dot(hm_ref[...], wr_ref[...], preferred_element_type=_F32)
        logits = logits + br_ref[...]
        mx = jnp.max(logits, axis=-1, keepdims=True)
        e = jnp.exp(logits - mx)
        w_s[...] = e / jnp.sum(e, axis=-1, keepdims=True)
        diff = jnp.abs(ur_ref[...] - mc_ref[...])
        same_b = (jax.lax.broadcasted_iota(_I32, (BM, BU), 0) // M
                  == jax.lax.broadcasted_iota(_I32, (BM, BU), 1) // U)
        adj = ((diff > 0) & (diff <= r_ref[0]) & same_b).astype(_F32)
        cnt = jnp.sum(adj, axis=-1, keepdims=True)
        adjn_s[...] = adj / jnp.maximum(cnt, 1.0)
        cpos_s[...] = (cnt > 0.0).astype(_F32)
        mix_s[...] = jnp.dot(w_s[...], b2_ref[...], preferred_element_type=_F32)

    @pl.when(p < K)
    def _expert():
        hid = jnp.dot(hub_s[...], w1_ref[0].astype(_BF16),
                      preferred_element_type=_F32)
        hid = _gelu_exact(hid + b1_ref[0]).astype(_BF16)
        sel = (jax.lax.broadcasted_iota(_I32, (1, K), 1) == p).astype(_F32)
        w_col = jnp.sum(w_s[...] * sel, axis=-1, keepdims=True)
        aw = (adjn_s[...] * w_col).astype(_BF16)
        t = jnp.dot(aw, hid, preferred_element_type=_F32).astype(_BF16)
        mix_s[...] += jnp.dot(t, w2_ref[0].astype(_BF16),
                              preferred_element_type=_F32)

    @pl.when(p == K - 1)
    def _fin():
        mix = mix_s[...]
        mu = jnp.mean(mix, axis=-1, keepdims=True)
        var = jnp.mean((mix - mu) ** 2, axis=-1, keepdims=True)
        mix_s[...] = (mix - mu) * jax.lax.rsqrt(var + 1e-5) * cpos_s[...]

    @pl.when(p >= K)
    def _scatter():
        q = p - K
        b = q // NS
        s = q % NS
        base = s * ST
        mr = mr_ref[0]
        nxt = jnp.concatenate([mr[:, 1:], jnp.full((1, 1), -1, _I32)], axis=1)
        last = mr != nxt
        col = jax.lax.broadcasted_iota(_I32, (ST, M), 0) + base
        pm = ((col == mr) & last).astype(_F32)
        mb = pl.multiple_of(b * M, M)
        out_ref[0] = jnp.dot(pm, mix_s[pl.ds(mb, M)],
                             preferred_element_type=_F32)


def kernel(h_L, mask_indices, unmasked_indices, range_r, W_r, b_r,
           W1, b1, W2, b2):
    B, S, D = h_L.shape
    M = mask_indices.shape[1]
    U = unmasked_indices.shape[1]
    K = W_r.shape[1]
    F = W1.shape[2]
    ST = 512
    NS = S // ST
    NSB = B * NS
    mi = mask_indices.astype(_I32)
    ui = unmasked_indices.astype(_I32)
    r_arr = jnp.asarray(range_r, _I32).reshape(1)

    offs = (jnp.arange(B, dtype=_I32) * S)[:, None]
    hu, hm = _sc_gather(h_L.reshape(B * S, D),
                        (ui + offs).reshape(-1), (mi + offs).reshape(-1))

    def w_idx(p):
        return (jnp.clip(p, 0, K - 1), 0, 0)

    def mr_idx(p):
        return (jnp.clip((p - K) // NS, 0, B - 1), 0, 0)

    def out_idx(p):
        q = jnp.clip(p - K, 0, NSB - 1)
        return (q // NS, q % NS, 0)

    out = pl.pallas_call(
        functools.partial(_tc_body, B=B, NS=NS, ST=ST, U=U, M=M, K=K),
        grid=(K + NSB,),
        in_specs=[
            pl.BlockSpec((B * U, D), lambda p: (0, 0)),
            pl.BlockSpec((B * M, D), lambda p: (0, 0)),
            pl.BlockSpec((B * M, 1), lambda p: (0, 0)),
            pl.BlockSpec((1, 1, M), mr_idx),
            pl.BlockSpec((1, B * U), lambda p: (0, 0)),
            pl.BlockSpec(memory_space=pltpu.SMEM),
            pl.BlockSpec((D, K), lambda p: (0, 0)),
            pl.BlockSpec((1, K), lambda p: (0, 0)),
            pl.BlockSpec((1, D, F), w_idx),
            pl.BlockSpec((1, 1, F), w_idx),
            pl.BlockSpec((1, F, D), w_idx),
            pl.BlockSpec((K, D), lambda p: (0, 0)),
        ],
        out_specs=pl.BlockSpec((1, ST, D), out_idx),
        out_shape=jax.ShapeDtypeStruct((B, S, D), _F32),
        scratch_shapes=[
            pltpu.VMEM((B * U, D), _BF16),
            pltpu.VMEM((B * M, K), _F32),
            pltpu.VMEM((B * M, B * U), _F32),
            pltpu.VMEM((B * M, 1), _F32),
            pltpu.VMEM((B * M, D), _F32),
        ],
    )(hu, hm, mi.reshape(B * M, 1), mi.reshape(B, 1, M),
      ui.reshape(1, B * U), r_arr,
      W_r, b_r.reshape(1, K), W1, b1.reshape(K, 1, F), W2, b2)
    return out
